# baseline (speedup 1.0000x reference)
import functools

import jax
import jax.numpy as jnp
from jax import lax
from jax.experimental import pallas as pl
from jax.experimental.pallas import tpu as pltpu
from jax.experimental.pallas import tpu_sc as plsc

N = 10000
E = 320000
NUM_GRAPHS = 64

NC = 2
NS = 16
NW = NC * NS
CHUNK = 128
TOT_ROWS = 2560
NBUF = 8
EPAD = TOT_ROWS * CHUNK
PROP_SPLIT = (120, 40)
DEG_SPLIT = (96, 64)
NP = 10016
NA = 10240
STRIPE = NA // NS

_MESH = plsc.VectorSubcoreMesh(core_axis_name="c", subcore_axis_name="s")
_SC_PARAMS = pltpu.CompilerParams(use_tc_tiling_on_sc=False)


def _make_prop(C):
    R0, R1 = PROP_SPLIT
    RMAX = max(R0, R1)

    @functools.partial(
        pl.kernel,
        out_type=jax.ShapeDtypeStruct((NC, NA, C), jnp.float32),
        mesh=_MESH,
        compiler_params=_SC_PARAMS,
        scratch_types=[
            pltpu.VMEM((RMAX, CHUNK), jnp.int32),
            pltpu.VMEM((RMAX, CHUNK), jnp.int32),
            pltpu.VMEM((NBUF, CHUNK, C), jnp.float32),
            pltpu.VMEM_SHARED((NA, C), jnp.float32),
            pltpu.VMEM_SHARED((NP, C), jnp.float32),
            pltpu.SemaphoreType.DMA((NBUF,)),
            pltpu.SemaphoreType.DMA((NBUF,)),
        ],
    )
    def prop(g_hbm, src_hbm, dst_hbm, z_hbm, out_hbm,
             src_v, dst_v, rows_v, acc, gtab, gsem, ssem):
        c = lax.axis_index("c")
        s = lax.axis_index("s")
        pltpu.sync_copy(z_hbm.at[pl.ds(s * STRIPE, STRIPE)],
                        acc.at[pl.ds(s * STRIPE, STRIPE)])
        pltpu.sync_copy(g_hbm.at[pl.ds(s * (NP // NS), NP // NS)],
                        gtab.at[pl.ds(s * (NP // NS), NP // NS)])

        @pl.when(c == 0)
        def _():
            pltpu.sync_copy(src_hbm.at[pl.ds(s * R0, R0)], src_v)
            pltpu.sync_copy(dst_hbm.at[pl.ds(s * R0, R0)], dst_v)

        @pl.when(c == 1)
        def _():
            base = NS * R0 + s * R1
            pltpu.sync_copy(src_hbm.at[pl.ds(base, R1)],
                            src_v.at[pl.ds(0, R1)])
            pltpu.sync_copy(dst_hbm.at[pl.ds(base, R1)],
                            dst_v.at[pl.ds(0, R1)])

        plsc.subcore_barrier()
        ngrp = jnp.where(c == 0, R0 // NBUF, R1 // NBUF)

        for b in range(NBUF):
            pltpu.async_copy(gtab.at[src_v.at[b]], rows_v.at[b], gsem.at[b])

        def body(grp, carry):
            for b in range(NBUF):
                j = grp * NBUF + b
                pltpu.make_async_copy(
                    gtab.at[src_v.at[j]], rows_v.at[b], gsem.at[b]).wait()
                pltpu.async_copy(rows_v.at[b], acc.at[dst_v.at[j]],
                                 ssem.at[b], add=True)

            @pl.when(grp + 1 < ngrp)
            def _():
                for b in range(NBUF):
                    j = grp * NBUF + b
                    pltpu.make_async_copy(
                        rows_v.at[b], acc.at[dst_v.at[j]], ssem.at[b]).wait()
                    pltpu.async_copy(gtab.at[src_v.at[j + NBUF]],
                                     rows_v.at[b], gsem.at[b])
            return carry

        lax.fori_loop(0, ngrp, body, 0)
        for b in range(NBUF):
            pltpu.make_async_copy(
                rows_v.at[b], acc.at[dst_v.at[(ngrp - 1) * NBUF + b]],
                ssem.at[b]).wait()
        plsc.subcore_barrier()
        pltpu.sync_copy(acc.at[pl.ds(s * STRIPE, STRIPE)],
                        out_hbm.at[c, pl.ds(s * STRIPE, STRIPE)])

    return prop


_prop16 = _make_prop(16)
_prop32 = _make_prop(32)

DEG_C = 16


@functools.partial(
    pl.kernel,
    out_type=jax.ShapeDtypeStruct((NC, NA, DEG_C), jnp.float32),
    mesh=_MESH,
    compiler_params=_SC_PARAMS,
    scratch_types=[
        pltpu.VMEM((max(DEG_SPLIT), CHUNK), jnp.int32),
        pltpu.VMEM((CHUNK, DEG_C), jnp.float32),
        pltpu.VMEM_SHARED((NA, DEG_C), jnp.float32),
        pltpu.SemaphoreType.DMA,
    ],
)
def _deg(ones_hbm, dst_hbm, z_hbm, out_hbm, dst_v, ones_v, acc, sem):
    c = lax.axis_index("c")
    s = lax.axis_index("s")
    D0, D1 = DEG_SPLIT
    pltpu.sync_copy(z_hbm.at[pl.ds(s * STRIPE, STRIPE)],
                    acc.at[pl.ds(s * STRIPE, STRIPE)])

    @pl.when(c == 0)
    def _():
        pltpu.sync_copy(dst_hbm.at[pl.ds(s * D0, D0)], dst_v)

    @pl.when(c == 1)
    def _():
        pltpu.sync_copy(dst_hbm.at[pl.ds(NS * D0 + s * D1, D1)],
                        dst_v.at[pl.ds(0, D1)])

    pltpu.sync_copy(ones_hbm.at[pl.ds(0, CHUNK)], ones_v)
    plsc.subcore_barrier()
    nrows = jnp.where(c == 0, D0, D1)

    def body(j, carry):
        pltpu.async_copy(ones_v, acc.at[dst_v.at[j]], sem, add=True)
        return carry

    lax.fori_loop(0, nrows, body, 0)

    def drain(j, carry):
        pltpu.make_async_copy(ones_v, acc.at[dst_v.at[0]], sem).wait()
        return carry

    lax.fori_loop(0, nrows, drain, 0)
    plsc.subcore_barrier()
    pltpu.sync_copy(acc.at[pl.ds(s * STRIPE, STRIPE)],
                    out_hbm.at[c, pl.ds(s * STRIPE, STRIPE)])


def _tc_mm1_body(x_ref, w_ref, m_ref):
    m_ref[...] = jnp.dot(x_ref[...], w_ref[...],
                         preferred_element_type=jnp.float32)


def _tc_scale1_body(m_ref, dp_ref, g_ref, dis_ref):
    deg = dp_ref[0, :N, :] + dp_ref[1, :N, :] + 1.0
    dis = lax.rsqrt(deg)
    g_ref[:N, :] = m_ref[...] * dis
    g_ref[N:, :] = jnp.zeros((NP - N, g_ref.shape[1]), jnp.float32)
    dis_ref[:N, :] = dis
    dis_ref[N:, :] = jnp.zeros((NP - N, 1), jnp.float32)


def _tc_mid_body(p_ref, gprev_ref, dis_ref, b_ref, w_ref, gnext_ref):
    agg = p_ref[0, :NP, :] + p_ref[1, :NP, :] + gprev_ref[...]
    h = jnp.maximum(agg * dis_ref[...] + b_ref[...], 0.0)
    gnext_ref[...] = (
        jnp.dot(h, w_ref[...], preferred_element_type=jnp.float32)
        * dis_ref[...])


def _tc_pool_body(p_ref, gprev_ref, dis_ref, b_ref, batch_ref, out_ref):
    agg = p_ref[0, :N, :] + p_ref[1, :N, :] + gprev_ref[:N, :]
    h = agg * dis_ref[:N, :] + b_ref[...]
    gids = lax.broadcasted_iota(jnp.int32, (N, NUM_GRAPHS), 1)
    oh = (batch_ref[...] == gids).astype(jnp.float32)
    hc = jnp.concatenate([h, jnp.ones((N, 1), jnp.float32)], axis=1)
    sums = lax.dot_general(oh, hc, (((0,), (0,)), ((), ())),
                           preferred_element_type=jnp.float32)
    out_ref[...] = sums[:, :16] / jnp.maximum(sums[:, 16:17], 1.0)


def _tc_mm1(x, W1):
    return pl.pallas_call(
        _tc_mm1_body,
        out_shape=jax.ShapeDtypeStruct((N, 32), jnp.float32),
    )(x, W1)


def _tc_scale1(m, dp):
    return pl.pallas_call(
        _tc_scale1_body,
        out_shape=[jax.ShapeDtypeStruct((NP, 32), jnp.float32),
                   jax.ShapeDtypeStruct((NP, 1), jnp.float32)],
    )(m, dp)


def _tc_mid(p, gprev, dis, b, Wnext, cout):
    return pl.pallas_call(
        _tc_mid_body,
        out_shape=jax.ShapeDtypeStruct((NP, cout), jnp.float32),
    )(p, gprev, dis, b.reshape(1, -1), Wnext)


def _tc_pool(p, gprev, dis, b, batch):
    return pl.pallas_call(
        _tc_pool_body,
        out_shape=jax.ShapeDtypeStruct((NUM_GRAPHS, 16), jnp.float32),
    )(p, gprev, dis, b.reshape(1, -1), batch.reshape(N, 1))


def kernel(x, edge_index, batch, W1, b1, W2, b2, W3, b3, W4, b4):
    pad = jnp.full((EPAD - E,), N, dtype=jnp.int32)
    src2d = jnp.concatenate([edge_index[0], pad]).reshape(EPAD // CHUNK, CHUNK)
    dst2d = jnp.concatenate([edge_index[1], pad]).reshape(EPAD // CHUNK, CHUNK)
    z16 = jnp.zeros((NA, 16), jnp.float32)
    z32 = jnp.zeros((NA, 32), jnp.float32)
    ones16 = jnp.ones((NP, 16), jnp.float32)

    dp = _deg(ones16, dst2d, z16)
    m1 = _tc_mm1(x, W1)
    g1, dis = _tc_scale1(m1, dp[:, :, 0:1])
    p1 = _prop32(g1, src2d, dst2d, z32)
    g2 = _tc_mid(p1, g1, dis, b1, W2, 16)
    p2 = _prop16(g2, src2d, dst2d, z16)
    g3 = _tc_mid(p2, g2, dis, b2, W3, 16)
    p3 = _prop16(g3, src2d, dst2d, z16)
    g4 = _tc_mid(p3, g3, dis, b3, W4, 16)
    p4 = _prop16(g4, src2d, dst2d, z16)
    return _tc_pool(p4, g4, dis, b4, batch)


# speedup vs baseline: 1.0921x; 1.0921x over previous
import functools

import jax
import jax.numpy as jnp
from jax import lax
from jax.experimental import pallas as pl
from jax.experimental.pallas import tpu as pltpu
from jax.experimental.pallas import tpu_sc as plsc

N = 10000
E = 320000
NUM_GRAPHS = 64

NC = 2
NS = 16
NW = NC * NS
CHUNK = 128
TOT_ROWS = 2560
NBUF = 8
EPAD = TOT_ROWS * CHUNK
PROP_SPLIT = (96, 64)
DEG_SPLIT = (96, 64)
NP = 10016
NA = 10240
STRIPE = NA // NS

_MESH = plsc.VectorSubcoreMesh(core_axis_name="c", subcore_axis_name="s")
_SC_PARAMS = pltpu.CompilerParams(use_tc_tiling_on_sc=False)


def _make_prop(C):
    R0, R1 = PROP_SPLIT
    RMAX = max(R0, R1)

    @functools.partial(
        pl.kernel,
        out_type=jax.ShapeDtypeStruct((NC, NA, C), jnp.float32),
        mesh=_MESH,
        compiler_params=_SC_PARAMS,
        scratch_types=[
            pltpu.VMEM((RMAX, CHUNK), jnp.int32),
            pltpu.VMEM((RMAX, CHUNK), jnp.int32),
            pltpu.VMEM((NBUF, CHUNK, C), jnp.float32),
            pltpu.VMEM_SHARED((NA, C), jnp.float32),
            pltpu.VMEM_SHARED((NP, C), jnp.float32),
            pltpu.SemaphoreType.DMA((NBUF,)),
            pltpu.SemaphoreType.DMA((NBUF,)),
        ],
    )
    def prop(g_hbm, src_hbm, dst_hbm, z_hbm, out_hbm,
             src_v, dst_v, rows_v, acc, gtab, gsem, ssem):
        c = lax.axis_index("c")
        s = lax.axis_index("s")
        pltpu.sync_copy(z_hbm.at[pl.ds(s * STRIPE, STRIPE)],
                        acc.at[pl.ds(s * STRIPE, STRIPE)])
        pltpu.sync_copy(g_hbm.at[pl.ds(s * (NP // NS), NP // NS)],
                        gtab.at[pl.ds(s * (NP // NS), NP // NS)])

        @pl.when(c == 0)
        def _():
            pltpu.sync_copy(src_hbm.at[pl.ds(s * R0, R0)], src_v)
            pltpu.sync_copy(dst_hbm.at[pl.ds(s * R0, R0)], dst_v)

        @pl.when(c == 1)
        def _():
            base = NS * R0 + s * R1
            pltpu.sync_copy(src_hbm.at[pl.ds(base, R1)],
                            src_v.at[pl.ds(0, R1)])
            pltpu.sync_copy(dst_hbm.at[pl.ds(base, R1)],
                            dst_v.at[pl.ds(0, R1)])

        plsc.subcore_barrier()
        ngrp = jnp.where(c == 0, R0 // NBUF, R1 // NBUF)

        for b in range(NBUF):
            pltpu.async_copy(gtab.at[src_v.at[b]], rows_v.at[b], gsem.at[b])

        def body(grp, carry):
            for b in range(NBUF):
                j = grp * NBUF + b
                pltpu.make_async_copy(
                    gtab.at[src_v.at[j]], rows_v.at[b], gsem.at[b]).wait()
                pltpu.async_copy(rows_v.at[b], acc.at[dst_v.at[j]],
                                 ssem.at[b], add=True)

            @pl.when(grp + 1 < ngrp)
            def _():
                for b in range(NBUF):
                    j = grp * NBUF + b
                    pltpu.make_async_copy(
                        rows_v.at[b], acc.at[dst_v.at[j]], ssem.at[b]).wait()
                    pltpu.async_copy(gtab.at[src_v.at[j + NBUF]],
                                     rows_v.at[b], gsem.at[b])
            return carry

        lax.fori_loop(0, ngrp, body, 0)
        for b in range(NBUF):
            pltpu.make_async_copy(
                rows_v.at[b], acc.at[dst_v.at[(ngrp - 1) * NBUF + b]],
                ssem.at[b]).wait()
        plsc.subcore_barrier()
        pltpu.sync_copy(acc.at[pl.ds(s * STRIPE, STRIPE)],
                        out_hbm.at[c, pl.ds(s * STRIPE, STRIPE)])

    return prop


_prop16 = _make_prop(16)
_prop32 = _make_prop(32)

DEG_C = 16


@functools.partial(
    pl.kernel,
    out_type=jax.ShapeDtypeStruct((NC, NA, DEG_C), jnp.float32),
    mesh=_MESH,
    compiler_params=_SC_PARAMS,
    scratch_types=[
        pltpu.VMEM((max(DEG_SPLIT), CHUNK), jnp.int32),
        pltpu.VMEM((CHUNK, DEG_C), jnp.float32),
        pltpu.VMEM_SHARED((NA, DEG_C), jnp.float32),
        pltpu.SemaphoreType.DMA,
    ],
)
def _deg(ones_hbm, dst_hbm, z_hbm, out_hbm, dst_v, ones_v, acc, sem):
    c = lax.axis_index("c")
    s = lax.axis_index("s")
    D0, D1 = DEG_SPLIT
    pltpu.sync_copy(z_hbm.at[pl.ds(s * STRIPE, STRIPE)],
                    acc.at[pl.ds(s * STRIPE, STRIPE)])

    @pl.when(c == 0)
    def _():
        pltpu.sync_copy(dst_hbm.at[pl.ds(s * D0, D0)], dst_v)

    @pl.when(c == 1)
    def _():
        pltpu.sync_copy(dst_hbm.at[pl.ds(NS * D0 + s * D1, D1)],
                        dst_v.at[pl.ds(0, D1)])

    pltpu.sync_copy(ones_hbm.at[pl.ds(0, CHUNK)], ones_v)
    plsc.subcore_barrier()
    nrows = jnp.where(c == 0, D0, D1)

    def body(j, carry):
        pltpu.async_copy(ones_v, acc.at[dst_v.at[j]], sem, add=True)
        return carry

    lax.fori_loop(0, nrows, body, 0)

    def drain(j, carry):
        pltpu.make_async_copy(ones_v, acc.at[dst_v.at[0]], sem).wait()
        return carry

    lax.fori_loop(0, nrows, drain, 0)
    plsc.subcore_barrier()
    pltpu.sync_copy(acc.at[pl.ds(s * STRIPE, STRIPE)],
                    out_hbm.at[c, pl.ds(s * STRIPE, STRIPE)])


def _tc_mm1_body(x_ref, w_ref, m_ref):
    m_ref[...] = jnp.dot(x_ref[...], w_ref[...],
                         preferred_element_type=jnp.float32)


def _tc_scale1_body(m_ref, dp_ref, g_ref, dis_ref):
    deg = dp_ref[0, :N, :] + dp_ref[1, :N, :] + 1.0
    dis = lax.rsqrt(deg)
    g_ref[:N, :] = m_ref[...] * dis
    g_ref[N:, :] = jnp.zeros((NP - N, g_ref.shape[1]), jnp.float32)
    dis_ref[:N, :] = dis
    dis_ref[N:, :] = jnp.zeros((NP - N, 1), jnp.float32)


def _tc_mid_body(p_ref, gprev_ref, dis_ref, b_ref, w_ref, gnext_ref):
    agg = p_ref[0, :NP, :] + p_ref[1, :NP, :] + gprev_ref[...]
    h = jnp.maximum(agg * dis_ref[...] + b_ref[...], 0.0)
    gnext_ref[...] = (
        jnp.dot(h, w_ref[...], preferred_element_type=jnp.float32)
        * dis_ref[...])


def _tc_pool_body(p_ref, gprev_ref, dis_ref, b_ref, batch_ref, out_ref):
    agg = p_ref[0, :N, :] + p_ref[1, :N, :] + gprev_ref[:N, :]
    h = agg * dis_ref[:N, :] + b_ref[...]
    gids = lax.broadcasted_iota(jnp.int32, (N, NUM_GRAPHS), 1)
    oh = (batch_ref[...] == gids).astype(jnp.float32)
    hc = jnp.concatenate([h, jnp.ones((N, 1), jnp.float32)], axis=1)
    sums = lax.dot_general(oh, hc, (((0,), (0,)), ((), ())),
                           preferred_element_type=jnp.float32)
    out_ref[...] = sums[:, :16] / jnp.maximum(sums[:, 16:17], 1.0)


def _tc_mm1(x, W1):
    return pl.pallas_call(
        _tc_mm1_body,
        out_shape=jax.ShapeDtypeStruct((N, 32), jnp.float32),
    )(x, W1)


def _tc_scale1(m, dp):
    return pl.pallas_call(
        _tc_scale1_body,
        out_shape=[jax.ShapeDtypeStruct((NP, 32), jnp.float32),
                   jax.ShapeDtypeStruct((NP, 1), jnp.float32)],
    )(m, dp)


def _tc_mid(p, gprev, dis, b, Wnext, cout):
    return pl.pallas_call(
        _tc_mid_body,
        out_shape=jax.ShapeDtypeStruct((NP, cout), jnp.float32),
    )(p, gprev, dis, b.reshape(1, -1), Wnext)


def _tc_pool(p, gprev, dis, b, batch):
    return pl.pallas_call(
        _tc_pool_body,
        out_shape=jax.ShapeDtypeStruct((NUM_GRAPHS, 16), jnp.float32),
    )(p, gprev, dis, b.reshape(1, -1), batch.reshape(N, 1))


def kernel(x, edge_index, batch, W1, b1, W2, b2, W3, b3, W4, b4):
    pad = jnp.full((EPAD - E,), N, dtype=jnp.int32)
    src2d = jnp.concatenate([edge_index[0], pad]).reshape(EPAD // CHUNK, CHUNK)
    dst2d = jnp.concatenate([edge_index[1], pad]).reshape(EPAD // CHUNK, CHUNK)
    z16 = jnp.zeros((NA, 16), jnp.float32)
    z32 = jnp.zeros((NA, 32), jnp.float32)
    ones16 = jnp.ones((NP, 16), jnp.float32)

    dp = _deg(ones16, dst2d, z16)
    m1 = _tc_mm1(x, W1)
    g1, dis = _tc_scale1(m1, dp[:, :, 0:1])
    p1 = _prop32(g1, src2d, dst2d, z32)
    g2 = _tc_mid(p1, g1, dis, b1, W2, 16)
    p2 = _prop16(g2, src2d, dst2d, z16)
    g3 = _tc_mid(p2, g2, dis, b2, W3, 16)
    p3 = _prop16(g3, src2d, dst2d, z16)
    g4 = _tc_mid(p3, g3, dis, b3, W4, 16)
    p4 = _prop16(g4, src2d, dst2d, z16)
    return _tc_pool(p4, g4, dis, b4, batch)


# speedup vs baseline: 1.4642x; 1.3407x over previous
import functools

import jax
import jax.numpy as jnp
from jax import lax
from jax.experimental import pallas as pl
from jax.experimental.pallas import tpu as pltpu
from jax.experimental.pallas import tpu_sc as plsc

N = 10000
E = 320000
NUM_GRAPHS = 64

NC = 2
NS = 16
CHUNK = 128
TOT_ROWS = 2560
NBUF = 8
EPAD = TOT_ROWS * CHUNK
PROP_SPLIT = (96, 64)
DEG_SPLIT = (96, 64)
NPAD = 10048
NA = 10240
STRIPE = NA // NS
GSTR = NPAD // NS
NPG = NPAD // 8
NAG = NA // 8
NRG = N // 8

_MESH = plsc.VectorSubcoreMesh(core_axis_name="c", subcore_axis_name="s")
_SC_PARAMS = pltpu.CompilerParams(use_tc_tiling_on_sc=False)
F32 = jnp.float32


def _stage_idx(src_hbm, dst_hbm, src_v, dst_v, c, s, R0, R1):
    @pl.when(c == 0)
    def _():
        pltpu.sync_copy(src_hbm.at[pl.ds(s * R0, R0)], src_v)
        pltpu.sync_copy(dst_hbm.at[pl.ds(s * R0, R0)], dst_v)

    @pl.when(c == 1)
    def _():
        base = NS * R0 + s * R1
        pltpu.sync_copy(src_hbm.at[pl.ds(base, R1)], src_v.at[pl.ds(0, R1)])
        pltpu.sync_copy(dst_hbm.at[pl.ds(base, R1)], dst_v.at[pl.ds(0, R1)])


def _make_prop(ntab):
    R0, R1 = PROP_SPLIT
    RMAX = max(R0, R1)
    out_t = [jax.ShapeDtypeStruct((NC, NA, 16), F32) for _ in range(ntab)]
    scratch = [
        pltpu.VMEM((RMAX, CHUNK), jnp.int32),
        pltpu.VMEM((RMAX, CHUNK), jnp.int32),
    ]
    for _ in range(ntab):
        scratch += [
            pltpu.VMEM((NBUF, CHUNK, 16), F32),
            pltpu.VMEM_SHARED((NA, 16), F32),
            pltpu.VMEM_SHARED((NPAD, 16), F32),
        ]
    scratch += [
        pltpu.SemaphoreType.DMA((NBUF,)),
        pltpu.SemaphoreType.DMA((NBUF,)),
    ]

    @functools.partial(pl.kernel, out_type=out_t, mesh=_MESH,
                       compiler_params=_SC_PARAMS, scratch_types=scratch)
    def prop(*refs):
        g_hbm = refs[:ntab]
        src_hbm, dst_hbm, z_hbm = refs[ntab:ntab + 3]
        out_hbm = refs[ntab + 3:ntab + 3 + ntab]
        src_v, dst_v = refs[ntab + 3 + ntab:ntab + 5 + ntab]
        per = refs[ntab + 5 + ntab:ntab + 5 + ntab + 3 * ntab]
        rows_v = per[0::3]
        acc = per[1::3]
        gtab = per[2::3]
        gsem, ssem = refs[-2:]
        c = lax.axis_index("c")
        s = lax.axis_index("s")
        for t in range(ntab):
            pltpu.sync_copy(z_hbm.at[pl.ds(s * STRIPE, STRIPE)],
                            acc[t].at[pl.ds(s * STRIPE, STRIPE)])
            pltpu.sync_copy(g_hbm[t].at[pl.ds(s * GSTR, GSTR)],
                            gtab[t].at[pl.ds(s * GSTR, GSTR)])
        _stage_idx(src_hbm, dst_hbm, src_v, dst_v, c, s, R0, R1)
        plsc.subcore_barrier()
        ngrp = jnp.where(c == 0, R0 // NBUF, R1 // NBUF)

        for b in range(NBUF):
            for t in range(ntab):
                pltpu.async_copy(gtab[t].at[src_v.at[b]], rows_v[t].at[b],
                                 gsem.at[b])

        def body(grp, carry):
            for b in range(NBUF):
                j = grp * NBUF + b
                for t in range(ntab):
                    pltpu.make_async_copy(gtab[t].at[src_v.at[j]],
                                          rows_v[t].at[b], gsem.at[b]).wait()
                    pltpu.async_copy(rows_v[t].at[b], acc[t].at[dst_v.at[j]],
                                     ssem.at[b], add=True)

            @pl.when(grp + 1 < ngrp)
            def _():
                for b in range(NBUF):
                    j = grp * NBUF + b
                    for t in range(ntab):
                        pltpu.make_async_copy(
                            rows_v[t].at[b], acc[t].at[dst_v.at[j]],
                            ssem.at[b]).wait()
                        pltpu.async_copy(gtab[t].at[src_v.at[j + NBUF]],
                                         rows_v[t].at[b], gsem.at[b])
            return carry

        lax.fori_loop(0, ngrp, body, 0)
        for b in range(NBUF):
            j = (ngrp - 1) * NBUF + b
            for t in range(ntab):
                pltpu.make_async_copy(rows_v[t].at[b], acc[t].at[dst_v.at[j]],
                                      ssem.at[b]).wait()
        plsc.subcore_barrier()
        for t in range(ntab):
            pltpu.sync_copy(acc[t].at[pl.ds(s * STRIPE, STRIPE)],
                            out_hbm[t].at[c, pl.ds(s * STRIPE, STRIPE)])

    return prop


_prop1 = _make_prop(1)
_prop2 = _make_prop(2)


@functools.partial(
    pl.kernel,
    out_type=jax.ShapeDtypeStruct((NC, NA, 16), F32),
    mesh=_MESH,
    compiler_params=_SC_PARAMS,
    scratch_types=[
        pltpu.VMEM((max(DEG_SPLIT), CHUNK), jnp.int32),
        pltpu.VMEM((CHUNK, 16), F32),
        pltpu.VMEM_SHARED((NA, 16), F32),
        pltpu.SemaphoreType.DMA,
    ],
)
def _deg(ones_hbm, dst_hbm, z_hbm, out_hbm, dst_v, ones_v, acc, sem):
    c = lax.axis_index("c")
    s = lax.axis_index("s")
    D0, D1 = DEG_SPLIT
    pltpu.sync_copy(z_hbm.at[pl.ds(s * STRIPE, STRIPE)],
                    acc.at[pl.ds(s * STRIPE, STRIPE)])
    _stage_idx(dst_hbm, dst_hbm, dst_v, dst_v, c, s, D0, D1)
    pltpu.sync_copy(ones_hbm.at[pl.ds(0, CHUNK)], ones_v)
    plsc.subcore_barrier()
    nrows = jnp.where(c == 0, D0, D1)

    def body(j, carry):
        pltpu.async_copy(ones_v, acc.at[dst_v.at[j]], sem, add=True)
        return carry

    lax.fori_loop(0, nrows, body, 0)

    def drain(j, carry):
        pltpu.make_async_copy(ones_v, acc.at[dst_v.at[0]], sem).wait()
        return carry

    lax.fori_loop(0, nrows, drain, 0)
    plsc.subcore_barrier()
    pltpu.sync_copy(acc.at[pl.ds(s * STRIPE, STRIPE)],
                    out_hbm.at[c, pl.ds(s * STRIPE, STRIPE)])


def _tc_mm1_body(x8_ref, wa_ref, wb_ref, ma_ref, mb_ref):
    x8 = x8_ref[...]
    ma_ref[...] = jnp.dot(x8, wa_ref[...], preferred_element_type=F32)
    mb_ref[...] = jnp.dot(x8, wb_ref[...], preferred_element_type=F32)


def _tc_scale1_body(dpg_ref, ma_ref, mb_ref, dis_ref, ga_ref, gb_ref):
    deg = dpg_ref[0, :, :] + dpg_ref[1, :, :] + 1.0
    dis = lax.rsqrt(deg)
    dis_ref[:NRG, :] = dis[:NRG, :]
    dis_ref[NRG:, :] = jnp.zeros((NAG - NRG, 128), F32)
    ga_ref[:NRG, :] = ma_ref[...] * dis[:NRG, :]
    ga_ref[NRG:, :] = jnp.zeros((NPG - NRG, 128), F32)
    gb_ref[:NRG, :] = mb_ref[...] * dis[:NRG, :]
    gb_ref[NRG:, :] = jnp.zeros((NPG - NRG, 128), F32)


def _tc_mid2_body(pa_ref, pb_ref, ga_ref, gb_ref, dis_ref, ba_ref, bb_ref,
                  wa_ref, wb_ref, out_ref):
    dis = dis_ref[:NPG, :]
    ha = jnp.maximum((pa_ref[0, :NPG, :] + pa_ref[1, :NPG, :] + ga_ref[...])
                     * dis + ba_ref[...], 0.0)
    hb = jnp.maximum((pb_ref[0, :NPG, :] + pb_ref[1, :NPG, :] + gb_ref[...])
                     * dis + bb_ref[...], 0.0)
    out_ref[...] = (jnp.dot(ha, wa_ref[...], preferred_element_type=F32)
                    + jnp.dot(hb, wb_ref[...], preferred_element_type=F32)
                    ) * dis


def _tc_mid_body(p_ref, g_ref, dis_ref, b_ref, w_ref, out_ref):
    dis = dis_ref[:NPG, :]
    h = jnp.maximum((p_ref[0, :NPG, :] + p_ref[1, :NPG, :] + g_ref[...])
                    * dis + b_ref[...], 0.0)
    out_ref[...] = jnp.dot(h, w_ref[...], preferred_element_type=F32) * dis


def _tc_pool_body(p_ref, gprev_ref, dis_ref, b_ref, batch_ref, out_ref):
    agg = p_ref[0, :N, :] + p_ref[1, :N, :] + gprev_ref[:N, :]
    h = agg * dis_ref[:N, :] + b_ref[...]
    gids = lax.broadcasted_iota(jnp.int32, (N, NUM_GRAPHS), 1)
    oh = (batch_ref[...] == gids).astype(F32)
    hc = jnp.concatenate([h, jnp.ones((N, 1), F32)], axis=1)
    sums = lax.dot_general(oh, hc, (((0,), (0,)), ((), ())),
                           preferred_element_type=F32)
    out_ref[...] = sums[:, :16] / jnp.maximum(sums[:, 16:17], 1.0)


def _pc(body, out_shapes, *args):
    return pl.pallas_call(body, out_shape=out_shapes)(*args)


def kernel(x, edge_index, batch, W1, b1, W2, b2, W3, b3, W4, b4):
    pad = jnp.full((EPAD - E,), N, dtype=jnp.int32)
    src2d = jnp.concatenate([edge_index[0], pad]).reshape(TOT_ROWS, CHUNK)
    dst2d = jnp.concatenate([edge_index[1], pad]).reshape(TOT_ROWS, CHUNK)
    z16 = jnp.zeros((NA, 16), F32)
    ones16 = jnp.ones((NPAD, 16), F32)
    eye8 = jnp.eye(8, dtype=F32)
    w1a = jnp.kron(eye8, W1[:, :16])
    w1b = jnp.kron(eye8, W1[:, 16:])
    w2a = jnp.kron(eye8, W2[:16, :])
    w2b = jnp.kron(eye8, W2[16:, :])
    w3k = jnp.kron(eye8, W3)
    w4k = jnp.kron(eye8, W4)
    b1at = jnp.tile(b1[:16], 8).reshape(1, 128)
    b1bt = jnp.tile(b1[16:], 8).reshape(1, 128)
    b2t = jnp.tile(b2, 8).reshape(1, 128)
    b3t = jnp.tile(b3, 8).reshape(1, 128)

    dp = _deg(ones16, dst2d, z16)
    x8 = x.reshape(NRG, 1024)
    ma, mb = _pc(_tc_mm1_body,
                 [jax.ShapeDtypeStruct((NRG, 128), F32)] * 2, x8, w1a, w1b)
    disG, g1a, g1b = _pc(
        _tc_scale1_body,
        [jax.ShapeDtypeStruct((NAG, 128), F32),
         jax.ShapeDtypeStruct((NPG, 128), F32),
         jax.ShapeDtypeStruct((NPG, 128), F32)],
        dp.reshape(NC, NAG, 128), ma, mb)
    pa, pb = _prop2(g1a.reshape(NPAD, 16), g1b.reshape(NPAD, 16),
                    src2d, dst2d, z16)
    g2 = _pc(_tc_mid2_body, jax.ShapeDtypeStruct((NPG, 128), F32),
             pa.reshape(NC, NAG, 128), pb.reshape(NC, NAG, 128),
             g1a, g1b, disG, b1at, b1bt, w2a, w2b)
    p2, = _prop1(g2.reshape(NPAD, 16), src2d, dst2d, z16)
    g3 = _pc(_tc_mid_body, jax.ShapeDtypeStruct((NPG, 128), F32),
             p2.reshape(NC, NAG, 128), g2, disG, b2t, w3k)
    p3, = _prop1(g3.reshape(NPAD, 16), src2d, dst2d, z16)
    g4 = _pc(_tc_mid_body, jax.ShapeDtypeStruct((NPG, 128), F32),
             p3.reshape(NC, NAG, 128), g3, disG, b3t, w4k)
    p4, = _prop1(g4.reshape(NPAD, 16), src2d, dst2d, z16)
    dis_nm = disG.reshape(NA, 16)[:NPAD, 0:1]
    return _pc(_tc_pool_body, jax.ShapeDtypeStruct((NUM_GRAPHS, 16), F32),
               p4, g4.reshape(NPAD, 16), dis_nm,
               b4.reshape(1, 16), batch.reshape(N, 1))
